# baseline (speedup 1.0000x reference)
import functools

import jax
import jax.numpy as jnp
from jax import lax
from jax.experimental import pallas as pl
from jax.experimental.pallas import tpu as pltpu
from jax.experimental.pallas import tpu_sc as plsc

N = 10000
E = 160000
F = 256
NCLASS = 64

NC = 2
NS = 16
LANES = 16

RPT = 320
NPH = NS * RPT
NPAD = NC * NPH
K = 64
CPB = 64
MB = CPB * K
EPAD = E + 2 * MB
ER = EPAD // K


def _segsum_body(x_hbm, adj_hbm, row_hbm, col_hbm, bounds_hbm, out_hbm,
                 acc, colv, roww, adjv, rows_a, rows_b, bnd, sem_a, sem_b):
    c = lax.axis_index("c")
    s = lax.axis_index("s")
    wid = c * NS + s
    nbase = wid * RPT

    pltpu.sync_copy(bounds_hbm, bnd)
    lo_e = plsc.load_gather(bnd, [jnp.full((LANES,), wid, jnp.int32)])[0]
    hi_e = plsc.load_gather(bnd, [jnp.full((LANES,), wid + 1, jnp.int32)])[0]

    def _zero(r, carry):
        for j in range(F // LANES):
            acc[r, pl.ds(j * LANES, LANES)] = jnp.zeros((LANES,), jnp.float32)
        return carry
    lax.fori_loop(0, RPT, _zero, 0)

    startrow = (lo_e // K) // 8 * 8
    nblocks = (jnp.maximum(hi_e - startrow * K, 0) + MB - 1) // MB

    bufs = ((rows_a, sem_a), (rows_b, sem_b))

    def _block(m, carry):
        brow = startrow + m * CPB
        pltpu.sync_copy(col_hbm.at[pl.ds(brow, CPB)], colv)
        pltpu.sync_copy(row_hbm.at[pl.ds(brow, CPB)], roww)
        pltpu.sync_copy(adj_hbm.at[pl.ds(brow, CPB)], adjv)
        def _mask(q, carry2):
            ebase = (brow + q) * K
            for j in range(K // LANES):
                eid = ebase + j * LANES + lax.iota(jnp.int32, LANES)
                w = adjv[q, pl.ds(j * LANES, LANES)]
                valid = (eid >= lo_e) & (eid < hi_e)
                adjv[q, pl.ds(j * LANES, LANES)] = jnp.where(valid, w, 0.0)
            return carry2
        lax.fori_loop(0, CPB, _mask, 0)

        lane_iota = lax.iota(jnp.int32, LANES)
        pltpu.async_copy(x_hbm.at[colv.at[0]], rows_a, sem_a)

        def _pair(p, carry2):
            for b in range(2):
                q = p * 2 + b
                buf, sem = bufs[b]
                nbuf, nsem = bufs[1 - b]
                pltpu.make_async_copy(x_hbm.at[colv.at[q]], buf, sem).wait()

                @pl.when(q + 1 < CPB)
                def _start_next():
                    pltpu.async_copy(x_hbm.at[colv.at[q + 1]], nbuf, nsem)

                _buf = buf

                @plsc.parallel_loop(0, K, step=1, unroll=2)
                def _acc_edge(e):
                    qv = jnp.full((LANES,), q, jnp.int32)
                    ev = jnp.full((LANES,), e, jnp.int32)
                    w16 = plsc.load_gather(adjv, [qv, ev])
                    dst16 = plsc.load_gather(roww, [qv, ev])
                    vs = [_buf[e, pl.ds(j * LANES, LANES)]
                          for j in range(F // (2 * LANES))]
                    for j in range(F // (2 * LANES)):
                        fa, fb = plsc.unpack(
                            plsc.bitcast(vs[j], jnp.bfloat16),
                            format=plsc.PackFormat.INTERLEAVED)
                        plsc.addupdate_scatter(
                            acc, [dst16, 2 * j * LANES + 2 * lane_iota],
                            fa * w16)
                        plsc.addupdate_scatter(
                            acc, [dst16, 2 * j * LANES + 2 * lane_iota + 1],
                            fb * w16)
            return carry2
        lax.fori_loop(0, CPB // 2, _pair, 0)
        return carry
    lax.fori_loop(0, nblocks, _block, 0)

    pltpu.sync_copy(acc, out_hbm.at[pl.ds(nbase, RPT)])


def _sc_segsum(x, adj_p, row_p, col_p, bounds):
    mesh = plsc.VectorSubcoreMesh(core_axis_name="c", subcore_axis_name="s")
    fn = pl.kernel(
        _segsum_body,
        out_type=jax.ShapeDtypeStruct((NPAD, F), jnp.float32),
        mesh=mesh,
        scratch_types=[
            pltpu.VMEM((RPT, F), jnp.float32),
            pltpu.VMEM((CPB, K), jnp.int32),
            pltpu.VMEM((CPB, K), jnp.int32),
            pltpu.VMEM((CPB, K), jnp.float32),
            pltpu.VMEM((K, F // 2), jnp.int32),
            pltpu.VMEM((K, F // 2), jnp.int32),
            pltpu.VMEM((64,), jnp.int32),
            pltpu.SemaphoreType.DMA,
            pltpu.SemaphoreType.DMA,
        ],
        compiler_params=pltpu.CompilerParams(needs_layout_passes=False),
    )
    return fn(x, adj_p, row_p, col_p, bounds)


BM = 2000


def _tc_layer_body(beta_ref, x_ref, agg_ref, ws_ref, bs_ref, wn_ref, bn_ref,
                   o_ref, ob_ref):
    hs = jnp.dot(x_ref[...], ws_ref[...], preferred_element_type=jnp.float32)
    hs = jnp.maximum(hs + bs_ref[...], 0.0)
    hn = jnp.dot(agg_ref[...], wn_ref[...], preferred_element_type=jnp.float32)
    hn = jnp.maximum(hn + bn_ref[...], 0.0)
    b = jax.nn.sigmoid(beta_ref[0, 0])
    res = b * hs + (1.0 - b) * hn
    o_ref[...] = res
    ob_ref[...] = res.astype(jnp.bfloat16)


def _tc_layer(beta_p, x, agg, Ws, bs, Wn, bn):
    return pl.pallas_call(
        _tc_layer_body,
        grid=(N // BM,),
        in_specs=[
            pl.BlockSpec((1, 1), lambda i: (0, 0)),
            pl.BlockSpec((BM, F), lambda i: (i, 0)),
            pl.BlockSpec((BM, F), lambda i: (i, 0)),
            pl.BlockSpec((F, F), lambda i: (0, 0)),
            pl.BlockSpec((1, F), lambda i: (0, 0)),
            pl.BlockSpec((F, F), lambda i: (0, 0)),
            pl.BlockSpec((1, F), lambda i: (0, 0)),
        ],
        out_specs=[pl.BlockSpec((BM, F), lambda i: (i, 0)),
                   pl.BlockSpec((BM, F), lambda i: (i, 0))],
        out_shape=[jax.ShapeDtypeStruct((N, F), jnp.float32),
                   jax.ShapeDtypeStruct((N, F), jnp.bfloat16)],
    )(beta_p, x, agg, Ws, bs, Wn, bn)


def _tc_final_body(beta_ref, x_ref, agg_ref, ws_ref, bs_ref, wn_ref, bn_ref,
                   wc_ref, bc_ref, o_ref):
    hs = jnp.dot(x_ref[...], ws_ref[...], preferred_element_type=jnp.float32)
    hs = jnp.maximum(hs + bs_ref[...], 0.0)
    hn = jnp.dot(agg_ref[...], wn_ref[...], preferred_element_type=jnp.float32)
    hn = jnp.maximum(hn + bn_ref[...], 0.0)
    b = jax.nn.sigmoid(beta_ref[0, 0])
    hp = b * hs + (1.0 - b) * hn
    logits = jnp.dot(hp, wc_ref[...], preferred_element_type=jnp.float32) + bc_ref[...]
    m = jnp.max(logits, axis=1, keepdims=True)
    lse = jnp.log(jnp.sum(jnp.exp(logits - m), axis=1, keepdims=True)) + m
    o_ref[...] = logits - lse


def _tc_final(beta_p, x, agg, Ws, bs, Wn, bn, Wc, bc):
    return pl.pallas_call(
        _tc_final_body,
        grid=(N // BM,),
        in_specs=[
            pl.BlockSpec((1, 1), lambda i: (0, 0)),
            pl.BlockSpec((BM, F), lambda i: (i, 0)),
            pl.BlockSpec((BM, F), lambda i: (i, 0)),
            pl.BlockSpec((F, F), lambda i: (0, 0)),
            pl.BlockSpec((1, F), lambda i: (0, 0)),
            pl.BlockSpec((F, F), lambda i: (0, 0)),
            pl.BlockSpec((1, F), lambda i: (0, 0)),
            pl.BlockSpec((F, NCLASS), lambda i: (0, 0)),
            pl.BlockSpec((1, NCLASS), lambda i: (0, 0)),
        ],
        out_specs=pl.BlockSpec((BM, NCLASS), lambda i: (i, 0)),
        out_shape=jax.ShapeDtypeStruct((N, NCLASS), jnp.float32),
    )(beta_p, x, agg, Ws, bs, Wn, bn, Wc, bc)


def kernel(feat, adj, row, col,
           W_self_0, b_self_0, W_neigh_0, b_neigh_0, beta_0,
           W_self_1, b_self_1, W_neigh_1, b_neigh_1, beta_1,
           W_cls, b_cls):
    row = row.astype(jnp.int32)
    col = col.astype(jnp.int32)
    npad = EPAD - E
    adj_p = jnp.concatenate([adj, jnp.zeros((npad,), jnp.float32)]).reshape(ER, K)
    row_p = (jnp.concatenate([row, jnp.full((npad,), NPAD - 1, jnp.int32)])
             % RPT).reshape(ER, K)
    col_p = jnp.concatenate([col, jnp.zeros((npad,), jnp.int32)]).reshape(ER, K)
    node_bounds = jnp.arange(NC * NS + 1, dtype=jnp.int32) * RPT
    bounds = jnp.searchsorted(row, node_bounds).astype(jnp.int32)
    bounds = jnp.concatenate(
        [bounds, jnp.full((64 - bounds.shape[0],), E, jnp.int32)])

    bs0 = b_self_0.reshape(1, F)
    bn0 = b_neigh_0.reshape(1, F)
    bs1 = b_self_1.reshape(1, F)
    bn1 = b_neigh_1.reshape(1, F)
    bc = b_cls.reshape(1, NCLASS)
    be0 = beta_0.reshape(1, 1)
    be1 = beta_1.reshape(1, 1)

    featb = lax.bitcast_convert_type(
        feat.astype(jnp.bfloat16).reshape(N, F // 2, 2), jnp.int32)
    agg0 = _sc_segsum(featb, adj_p, row_p, col_p, bounds)
    x1, x1b = _tc_layer(be0, feat, agg0, W_self_0, bs0, W_neigh_0, bn0)
    x1p = lax.bitcast_convert_type(x1b.reshape(N, F // 2, 2), jnp.int32)
    agg1 = _sc_segsum(x1p, adj_p, row_p, col_p, bounds)
    return _tc_final(be1, x1, agg1, W_self_1, bs1, W_neigh_1, bn1, W_cls, bc)


# speedup vs baseline: 2.1959x; 2.1959x over previous
import functools

import jax
import jax.numpy as jnp
from jax import lax
from jax.experimental import pallas as pl
from jax.experimental.pallas import tpu as pltpu
from jax.experimental.pallas import tpu_sc as plsc

N = 10000
E = 160000
F = 256
NCLASS = 64

NC = 2
NS = 16
LANES = 16

RPT = 320
NPH = NS * RPT
NPAD = NC * NPH
K = 32
CPB = 48
MB = CPB * K
EPAD = E + 2 * MB
ER = EPAD // K


def _segsum_body(x_hbm, adj_hbm, row_hbm, col_hbm, bounds_hbm, out_hbm,
                 acc, colv, roww, adjv, rows_a, rows_b, rows_c, bnd,
                 sem_a, sem_b, sem_c):
    c = lax.axis_index("c")
    s = lax.axis_index("s")
    wid = c * NS + s
    nbase = wid * RPT

    pltpu.sync_copy(bounds_hbm, bnd)
    lo_e = plsc.load_gather(bnd, [jnp.full((LANES,), wid, jnp.int32)])[0]
    hi_e = plsc.load_gather(bnd, [jnp.full((LANES,), wid + 1, jnp.int32)])[0]

    def _zero(r, carry):
        for j in range(F // LANES):
            acc[r, pl.ds(j * LANES, LANES)] = jnp.zeros((LANES,), jnp.float32)
        return carry
    lax.fori_loop(0, RPT, _zero, 0)

    startrow = (lo_e // K) // 8 * 8
    nblocks = (jnp.maximum(hi_e - startrow * K, 0) + MB - 1) // MB

    bufs = ((rows_a, sem_a), (rows_b, sem_b), (rows_c, sem_c))

    def _block(m, carry):
        brow = startrow + m * CPB
        pltpu.sync_copy(col_hbm.at[pl.ds(brow, CPB)], colv)
        pltpu.sync_copy(row_hbm.at[pl.ds(brow, CPB)], roww)
        pltpu.sync_copy(adj_hbm.at[pl.ds(brow, CPB)], adjv)
        def _mask(q, carry2):
            ebase = (brow + q) * K
            for j in range(K // LANES):
                eid = ebase + j * LANES + lax.iota(jnp.int32, LANES)
                w = adjv[q, pl.ds(j * LANES, LANES)]
                valid = (eid >= lo_e) & (eid < hi_e)
                adjv[q, pl.ds(j * LANES, LANES)] = jnp.where(valid, w, 0.0)
            return carry2
        lax.fori_loop(0, CPB, _mask, 0)

        lane_iota = lax.iota(jnp.int32, LANES)
        for pr in range(2):
            pltpu.async_copy(x_hbm.at[colv.at[pr]], bufs[pr][0], bufs[pr][1])

        def _pair(p, carry2):
            for b in range(3):
                q = p * 3 + b
                buf, sem = bufs[b]
                nbuf, nsem = bufs[(b + 2) % 3]
                pltpu.make_async_copy(x_hbm.at[colv.at[q]], buf, sem).wait()

                @pl.when(q + 2 < CPB)
                def _start_next():
                    pltpu.async_copy(x_hbm.at[colv.at[q + 2]], nbuf, nsem)

                _buf = buf

                @plsc.parallel_loop(0, K, step=1, unroll=2)
                def _acc_edge(e):
                    qv = jnp.full((LANES,), q, jnp.int32)
                    ev = jnp.full((LANES,), e, jnp.int32)
                    w16 = plsc.load_gather(adjv, [qv, ev])
                    dst16 = plsc.load_gather(roww, [qv, ev])
                    vs = [_buf[e, pl.ds(j * LANES, LANES)]
                          for j in range(F // LANES)]
                    for j in range(F // LANES):
                        plsc.addupdate_scatter(
                            acc, [dst16, j * LANES + lane_iota], vs[j] * w16)
            return carry2
        lax.fori_loop(0, CPB // 3, _pair, 0)
        return carry
    lax.fori_loop(0, nblocks, _block, 0)

    pltpu.sync_copy(acc, out_hbm.at[pl.ds(nbase, RPT)])


def _sc_segsum(x, adj_p, row_p, col_p, bounds):
    mesh = plsc.VectorSubcoreMesh(core_axis_name="c", subcore_axis_name="s")
    fn = pl.kernel(
        _segsum_body,
        out_type=jax.ShapeDtypeStruct((NPAD, F), jnp.float32),
        mesh=mesh,
        scratch_types=[
            pltpu.VMEM((RPT, F), jnp.float32),
            pltpu.VMEM((CPB, K), jnp.int32),
            pltpu.VMEM((CPB, K), jnp.int32),
            pltpu.VMEM((CPB, K), jnp.float32),
            pltpu.VMEM((K, F), jnp.float32),
            pltpu.VMEM((K, F), jnp.float32),
            pltpu.VMEM((K, F), jnp.float32),
            pltpu.VMEM((64,), jnp.int32),
            pltpu.SemaphoreType.DMA,
            pltpu.SemaphoreType.DMA,
            pltpu.SemaphoreType.DMA,
        ],
        compiler_params=pltpu.CompilerParams(needs_layout_passes=False),
    )
    return fn(x, adj_p, row_p, col_p, bounds)


BM = 2000


def _tc_layer_body(beta_ref, x_ref, agg_ref, ws_ref, bs_ref, wn_ref, bn_ref, o_ref):
    hs = jnp.dot(x_ref[...], ws_ref[...], preferred_element_type=jnp.float32)
    hs = jnp.maximum(hs + bs_ref[...], 0.0)
    hn = jnp.dot(agg_ref[...], wn_ref[...], preferred_element_type=jnp.float32)
    hn = jnp.maximum(hn + bn_ref[...], 0.0)
    b = jax.nn.sigmoid(beta_ref[0, 0])
    o_ref[...] = b * hs + (1.0 - b) * hn


def _tc_layer(beta_p, x, agg, Ws, bs, Wn, bn):
    return pl.pallas_call(
        _tc_layer_body,
        grid=(N // BM,),
        in_specs=[
            pl.BlockSpec((1, 1), lambda i: (0, 0)),
            pl.BlockSpec((BM, F), lambda i: (i, 0)),
            pl.BlockSpec((BM, F), lambda i: (i, 0)),
            pl.BlockSpec((F, F), lambda i: (0, 0)),
            pl.BlockSpec((1, F), lambda i: (0, 0)),
            pl.BlockSpec((F, F), lambda i: (0, 0)),
            pl.BlockSpec((1, F), lambda i: (0, 0)),
        ],
        out_specs=pl.BlockSpec((BM, F), lambda i: (i, 0)),
        out_shape=jax.ShapeDtypeStruct((N, F), jnp.float32),
    )(beta_p, x, agg, Ws, bs, Wn, bn)


def _tc_final_body(beta_ref, x_ref, agg_ref, ws_ref, bs_ref, wn_ref, bn_ref,
                   wc_ref, bc_ref, o_ref):
    hs = jnp.dot(x_ref[...], ws_ref[...], preferred_element_type=jnp.float32)
    hs = jnp.maximum(hs + bs_ref[...], 0.0)
    hn = jnp.dot(agg_ref[...], wn_ref[...], preferred_element_type=jnp.float32)
    hn = jnp.maximum(hn + bn_ref[...], 0.0)
    b = jax.nn.sigmoid(beta_ref[0, 0])
    hp = b * hs + (1.0 - b) * hn
    logits = jnp.dot(hp, wc_ref[...], preferred_element_type=jnp.float32) + bc_ref[...]
    m = jnp.max(logits, axis=1, keepdims=True)
    lse = jnp.log(jnp.sum(jnp.exp(logits - m), axis=1, keepdims=True)) + m
    o_ref[...] = logits - lse


def _tc_final(beta_p, x, agg, Ws, bs, Wn, bn, Wc, bc):
    return pl.pallas_call(
        _tc_final_body,
        grid=(N // BM,),
        in_specs=[
            pl.BlockSpec((1, 1), lambda i: (0, 0)),
            pl.BlockSpec((BM, F), lambda i: (i, 0)),
            pl.BlockSpec((BM, F), lambda i: (i, 0)),
            pl.BlockSpec((F, F), lambda i: (0, 0)),
            pl.BlockSpec((1, F), lambda i: (0, 0)),
            pl.BlockSpec((F, F), lambda i: (0, 0)),
            pl.BlockSpec((1, F), lambda i: (0, 0)),
            pl.BlockSpec((F, NCLASS), lambda i: (0, 0)),
            pl.BlockSpec((1, NCLASS), lambda i: (0, 0)),
        ],
        out_specs=pl.BlockSpec((BM, NCLASS), lambda i: (i, 0)),
        out_shape=jax.ShapeDtypeStruct((N, NCLASS), jnp.float32),
    )(beta_p, x, agg, Ws, bs, Wn, bn, Wc, bc)


def kernel(feat, adj, row, col,
           W_self_0, b_self_0, W_neigh_0, b_neigh_0, beta_0,
           W_self_1, b_self_1, W_neigh_1, b_neigh_1, beta_1,
           W_cls, b_cls):
    row = row.astype(jnp.int32)
    col = col.astype(jnp.int32)
    npad = EPAD - E
    adj_p = jnp.concatenate([adj, jnp.zeros((npad,), jnp.float32)]).reshape(ER, K)
    row_p = (jnp.concatenate([row, jnp.full((npad,), NPAD - 1, jnp.int32)])
             % RPT).reshape(ER, K)
    col_p = jnp.concatenate([col, jnp.zeros((npad,), jnp.int32)]).reshape(ER, K)
    node_bounds = jnp.arange(NC * NS + 1, dtype=jnp.int32) * RPT
    bounds = jnp.searchsorted(row, node_bounds).astype(jnp.int32)
    bounds = jnp.concatenate(
        [bounds, jnp.full((64 - bounds.shape[0],), E, jnp.int32)])

    bs0 = b_self_0.reshape(1, F)
    bn0 = b_neigh_0.reshape(1, F)
    bs1 = b_self_1.reshape(1, F)
    bn1 = b_neigh_1.reshape(1, F)
    bc = b_cls.reshape(1, NCLASS)
    be0 = beta_0.reshape(1, 1)
    be1 = beta_1.reshape(1, 1)

    agg0 = _sc_segsum(feat, adj_p, row_p, col_p, bounds)
    x1 = _tc_layer(be0, feat, agg0, W_self_0, bs0, W_neigh_0, bn0)
    agg1 = _sc_segsum(x1, adj_p, row_p, col_p, bounds)
    return _tc_final(be1, x1, agg1, W_self_1, bs1, W_neigh_1, bn1, W_cls, bc)
